# baseline (speedup 1.0000x reference)
import functools

import jax
import jax.numpy as jnp
from jax import lax
from jax.experimental import pallas as pl
from jax.experimental.pallas import tpu as pltpu
from jax.experimental.pallas import tpu_sc as plsc

VOCAB = 1000
VPAD = 1024
N_EMBD = 64
T = 20
TPAD = 128
B = 1024

_NC = 2
_NS = 16
_NW = _NC * _NS
_VPT = VPAD // _NW
_GRP = 4
_NG = _VPT // _GRP


def _tw_body(tok_ref, pos_ref, w_ref, b_ref, twt_ref, pwt_ref):
    w = w_ref[...]
    twt_ref[...] = lax.dot_general(
        w, tok_ref[...],
        dimension_numbers=(((0,), (1,)), ((), ())),
        preferred_element_type=jnp.float32,
    )
    pwt_ref[...] = lax.dot_general(
        w, pos_ref[...],
        dimension_numbers=(((0,), (1,)), ((), ())),
        preferred_element_type=jnp.float32,
    ) + b_ref[...]


def _tc_tables(tok_pad, pos_pad, W_pv, b_col):
    return pl.pallas_call(
        _tw_body,
        grid=(1,),
        in_specs=[
            pl.BlockSpec((VPAD, N_EMBD), lambda i: (0, 0)),
            pl.BlockSpec((TPAD, N_EMBD), lambda i: (0, 0)),
            pl.BlockSpec((N_EMBD, VPAD), lambda i: (0, 0)),
            pl.BlockSpec((VPAD, 1), lambda i: (0, 0)),
        ],
        out_specs=[
            pl.BlockSpec((VPAD, VPAD), lambda i: (0, 0)),
            pl.BlockSpec((VPAD, TPAD), lambda i: (0, 0)),
        ],
        out_shape=[
            jax.ShapeDtypeStruct((VPAD, VPAD), jnp.float32),
            jax.ShapeDtypeStruct((VPAD, TPAD), jnp.float32),
        ],
    )(tok_pad, pos_pad, W_pv, b_col)


@functools.partial(
    pl.kernel,
    mesh=plsc.VectorSubcoreMesh(core_axis_name="c", subcore_axis_name="s"),
    out_type=jax.ShapeDtypeStruct((T, VOCAB, B), jnp.float32),
    scratch_types=[
        pltpu.VMEM((T * B,), jnp.int32),
        pltpu.VMEM((_VPT * TPAD,), jnp.float32),
        pltpu.VMEM((_GRP * VPAD,), jnp.float32),
        [pltpu.VMEM((T, B), jnp.float32) for _ in range(_GRP)],
        [pltpu.SemaphoreType.DMA for _ in range(_GRP)],
    ],
    compiler_params=pltpu.CompilerParams(use_tc_tiling_on_sc=False,
                                         needs_layout_passes=False),
)
def _sc_emit(idx_hbm, twt_hbm, pwt_hbm, out_hbm, idx_v, pw_s, twt_g, bufs,
             sems):
    wid = lax.axis_index("s") * _NC + lax.axis_index("c")
    vlo = wid * _VPT
    pltpu.sync_copy(idx_hbm, idx_v)
    pltpu.sync_copy(pwt_hbm.at[pl.ds(vlo * TPAD, _VPT * TPAD)], pw_s)

    @pl.loop(0, _NG)
    def _group(g):
        pltpu.sync_copy(
            twt_hbm.at[pl.ds((vlo + g * _GRP) * VPAD, _GRP * VPAD)], twt_g)
        for j in range(_GRP):
            v_loc = g * _GRP + j
            v_glob = vlo + v_loc
            buf = bufs[j]

            @pl.when((g > 0) & (v_glob - _GRP < VOCAB))
            def _():
                pltpu.make_async_copy(buf, out_hbm.at[:, 0, :], sems[j]).wait()

            jbase = jnp.full((16,), j * VPAD, jnp.int32)

            def t_body(t, _, jbase=jbase, buf=buf, v_loc=v_loc):
                pwv = plsc.load_gather(
                    pw_s, [jnp.full((16,), v_loc * TPAD, jnp.int32) + t])
                for c in range(B // 16):
                    idxv = idx_v[pl.ds(t * B + c * 16, 16)]
                    gv = plsc.load_gather(twt_g, [jbase + idxv])
                    buf[t, pl.ds(c * 16, 16)] = gv + pwv
                return _

            lax.fori_loop(0, T, t_body, None)

            @pl.when(v_glob < VOCAB)
            def _():
                pltpu.make_async_copy(buf, out_hbm.at[:, v_glob, :],
                                      sems[j]).start()

    for j in range(_GRP):
        @pl.when(vlo + (_NG - 1) * _GRP + j < VOCAB)
        def _(j=j):
            pltpu.make_async_copy(bufs[j], out_hbm.at[:, 0, :], sems[j]).wait()


def kernel(indices, tok_table, pos_table, W, b):
    Bv, Tv = indices.shape
    idx_tmaj = indices.T.reshape(-1).astype(jnp.int32)
    tok_pad = jnp.pad(tok_table, ((0, VPAD - VOCAB), (0, 0)))
    pos_pad = jnp.pad(pos_table[:Tv], ((0, TPAD - Tv), (0, 0)))
    W_pv = jnp.pad(W, ((0, 0), (0, VPAD - VOCAB)))
    b_col = jnp.pad(b, (0, VPAD - VOCAB)).reshape(VPAD, 1)
    twt, pwt = _tc_tables(tok_pad, pos_pad, W_pv, b_col)
    out3 = _sc_emit(idx_tmaj, twt.reshape(-1), pwt.reshape(-1))
    return jnp.transpose(out3, (2, 0, 1))


# speedup vs baseline: 2.9548x; 2.9548x over previous
import functools

import jax
import jax.numpy as jnp
from jax import lax
from jax.experimental import pallas as pl
from jax.experimental.pallas import tpu as pltpu
from jax.experimental.pallas import tpu_sc as plsc

VOCAB = 1000
VPAD = 1024
N_EMBD = 64
T = 20
TPAD = 128
B = 1024

_NC = 2
_NS = 16
_NW = _NC * _NS
_VPT = VPAD // _NW
_GRP = 4
_NG = _VPT // _GRP


def _tw_body(tok_ref, pos_ref, w_ref, b_ref, twt_ref, pwt_ref):
    w = w_ref[...]
    twt_ref[...] = lax.dot_general(
        w, tok_ref[...],
        dimension_numbers=(((0,), (1,)), ((), ())),
        preferred_element_type=jnp.float32,
    )
    pwt_ref[...] = lax.dot_general(
        w, pos_ref[...],
        dimension_numbers=(((0,), (1,)), ((), ())),
        preferred_element_type=jnp.float32,
    ) + b_ref[...]


def _tc_tables(tok_pad, pos_pad, W_pv, b_col):
    return pl.pallas_call(
        _tw_body,
        grid=(1,),
        in_specs=[
            pl.BlockSpec((VPAD, N_EMBD), lambda i: (0, 0)),
            pl.BlockSpec((TPAD, N_EMBD), lambda i: (0, 0)),
            pl.BlockSpec((N_EMBD, VPAD), lambda i: (0, 0)),
            pl.BlockSpec((VPAD, 1), lambda i: (0, 0)),
        ],
        out_specs=[
            pl.BlockSpec((VPAD, VPAD), lambda i: (0, 0)),
            pl.BlockSpec((VPAD, TPAD), lambda i: (0, 0)),
        ],
        out_shape=[
            jax.ShapeDtypeStruct((VPAD, VPAD), jnp.float32),
            jax.ShapeDtypeStruct((VPAD, TPAD), jnp.float32),
        ],
    )(tok_pad, pos_pad, W_pv, b_col)


@functools.partial(
    pl.kernel,
    mesh=plsc.VectorSubcoreMesh(core_axis_name="c", subcore_axis_name="s"),
    out_type=jax.ShapeDtypeStruct((T, VOCAB, B), jnp.float32),
    scratch_types=[
        pltpu.VMEM((T * B,), jnp.int32),
        pltpu.VMEM((_VPT * TPAD,), jnp.float32),
        pltpu.VMEM((_GRP * VPAD,), jnp.float32),
        [pltpu.VMEM((T, B), jnp.float32) for _ in range(_GRP)],
        [pltpu.SemaphoreType.DMA for _ in range(_GRP)],
    ],
    compiler_params=pltpu.CompilerParams(use_tc_tiling_on_sc=False,
                                         needs_layout_passes=False),
)
def _sc_emit(idx_hbm, twt_hbm, pwt_hbm, out_hbm, idx_v, pw_s, twt_g, bufs,
             sems):
    wid = lax.axis_index("s") * _NC + lax.axis_index("c")
    vlo = wid * _VPT
    pltpu.sync_copy(idx_hbm, idx_v)
    pltpu.sync_copy(pwt_hbm.at[pl.ds(vlo * TPAD, _VPT * TPAD)], pw_s)

    @pl.loop(0, _NG)
    def _group(g):
        pltpu.sync_copy(
            twt_hbm.at[pl.ds((vlo + g * _GRP) * VPAD, _GRP * VPAD)], twt_g)
        for j in range(_GRP):
            v_loc = g * _GRP + j
            v_glob = vlo + v_loc
            buf = bufs[j]

            @pl.when((g > 0) & (v_glob - _GRP < VOCAB))
            def _():
                pltpu.make_async_copy(buf, out_hbm.at[:, 0, :], sems[j]).wait()

            jbase = jnp.full((16,), j * VPAD, jnp.int32)

            def t_body(t, _, jbase=jbase, buf=buf, v_loc=v_loc):
                pwv = plsc.load_gather(
                    pw_s, [jnp.full((16,), v_loc * TPAD, jnp.int32) + t])

                @plsc.parallel_loop(0, B // 16, unroll=8)
                def _cols(c):
                    idxv = idx_v[pl.ds(t * B + c * 16, 16)]
                    gv = plsc.load_gather(twt_g, [jbase + idxv])
                    buf[t, pl.ds(c * 16, 16)] = gv + pwv

                return _

            lax.fori_loop(0, T, t_body, None)

            @pl.when(v_glob < VOCAB)
            def _():
                pltpu.make_async_copy(buf, out_hbm.at[:, v_glob, :],
                                      sems[j]).start()

    for j in range(_GRP):
        @pl.when(vlo + (_NG - 1) * _GRP + j < VOCAB)
        def _(j=j):
            pltpu.make_async_copy(bufs[j], out_hbm.at[:, 0, :], sems[j]).wait()


def kernel(indices, tok_table, pos_table, W, b):
    Bv, Tv = indices.shape
    idx_tmaj = indices.T.reshape(-1).astype(jnp.int32)
    tok_pad = jnp.pad(tok_table, ((0, VPAD - VOCAB), (0, 0)))
    pos_pad = jnp.pad(pos_table[:Tv], ((0, TPAD - Tv), (0, 0)))
    W_pv = jnp.pad(W, ((0, 0), (0, VPAD - VOCAB)))
    b_col = jnp.pad(b, (0, VPAD - VOCAB)).reshape(VPAD, 1)
    twt, pwt = _tc_tables(tok_pad, pos_pad, W_pv, b_col)
    out3 = _sc_emit(idx_tmaj, twt.reshape(-1), pwt.reshape(-1))
    return jnp.transpose(out3, (2, 0, 1))


# speedup vs baseline: 3.0680x; 1.0383x over previous
import functools

import jax
import jax.numpy as jnp
from jax import lax
from jax.experimental import pallas as pl
from jax.experimental.pallas import tpu as pltpu
from jax.experimental.pallas import tpu_sc as plsc

VOCAB = 1000
VPAD = 1024
N_EMBD = 64
T = 20
TPAD = 128
B = 1024

_NC = 2
_NS = 16
_NW = _NC * _NS
_VPT = VPAD // _NW
_GRP = 4
_NG = _VPT // _GRP


def _tw_body(tok_ref, pos_ref, w_ref, b_ref, twt_ref, pwt_ref):
    w = w_ref[...]
    twt_ref[...] = lax.dot_general(
        w, tok_ref[...],
        dimension_numbers=(((0,), (1,)), ((), ())),
        preferred_element_type=jnp.float32,
    )
    pwt_ref[...] = lax.dot_general(
        w, pos_ref[...],
        dimension_numbers=(((0,), (1,)), ((), ())),
        preferred_element_type=jnp.float32,
    ) + b_ref[...]


def _tc_tables(tok_pad, pos_pad, W_pv, b_col):
    return pl.pallas_call(
        _tw_body,
        grid=(1,),
        in_specs=[
            pl.BlockSpec((VPAD, N_EMBD), lambda i: (0, 0)),
            pl.BlockSpec((TPAD, N_EMBD), lambda i: (0, 0)),
            pl.BlockSpec((N_EMBD, VPAD), lambda i: (0, 0)),
            pl.BlockSpec((VPAD, 1), lambda i: (0, 0)),
        ],
        out_specs=[
            pl.BlockSpec((VPAD, VPAD), lambda i: (0, 0)),
            pl.BlockSpec((VPAD, TPAD), lambda i: (0, 0)),
        ],
        out_shape=[
            jax.ShapeDtypeStruct((VPAD, VPAD), jnp.float32),
            jax.ShapeDtypeStruct((VPAD, TPAD), jnp.float32),
        ],
    )(tok_pad, pos_pad, W_pv, b_col)


@functools.partial(
    pl.kernel,
    mesh=plsc.VectorSubcoreMesh(core_axis_name="c", subcore_axis_name="s"),
    out_type=jax.ShapeDtypeStruct((T, VOCAB, B), jnp.float32),
    scratch_types=[
        pltpu.VMEM((T * B,), jnp.int32),
        pltpu.VMEM((_VPT * TPAD,), jnp.float32),
        pltpu.VMEM((_VPT * VPAD,), jnp.float32),
        [pltpu.VMEM((_VPT, B), jnp.float32) for _ in range(2)],
        [pltpu.SemaphoreType.DMA for _ in range(2)],
    ],
    compiler_params=pltpu.CompilerParams(use_tc_tiling_on_sc=False,
                                         needs_layout_passes=False),
)
def _sc_emit(idx_hbm, twt_hbm, pwt_hbm, out_hbm, idx_v, pw_s, twt_all, bufs,
             sems):
    wid = lax.axis_index("s") * _NC + lax.axis_index("c")
    vlo = wid * _VPT
    nv_full = VOCAB - (_NW - 1) * _VPT
    pltpu.sync_copy(idx_hbm, idx_v)
    pltpu.sync_copy(pwt_hbm.at[pl.ds(vlo * TPAD, _VPT * TPAD)], pw_s)
    pltpu.sync_copy(twt_hbm.at[pl.ds(vlo * VPAD, _VPT * VPAD)], twt_all)

    def _start(buf, t):
        @pl.when(wid < _NW - 1)
        def _():
            pltpu.make_async_copy(buf, out_hbm.at[t, pl.ds(vlo, _VPT)],
                                  sems_of(buf)).start()

        @pl.when(wid == _NW - 1)
        def _():
            pltpu.make_async_copy(buf.at[pl.ds(0, nv_full)],
                                  out_hbm.at[t, pl.ds(vlo, nv_full)],
                                  sems_of(buf)).start()

    def _wait(buf):
        @pl.when(wid < _NW - 1)
        def _():
            pltpu.make_async_copy(buf, out_hbm.at[0, pl.ds(0, _VPT)],
                                  sems_of(buf)).wait()

        @pl.when(wid == _NW - 1)
        def _():
            pltpu.make_async_copy(buf.at[pl.ds(0, nv_full)],
                                  out_hbm.at[0, pl.ds(0, nv_full)],
                                  sems_of(buf)).wait()

    def sems_of(buf):
        return sems[0] if buf is bufs[0] else sems[1]

    @pl.loop(0, T, step=2)
    def _tpair(tt):
        for k in range(2):
            t = tt + k
            buf = bufs[k]

            @pl.when(tt > 0)
            def _(buf=buf):
                _wait(buf)

            def v_body(v, _, buf=buf, t=t):
                pwv = plsc.load_gather(
                    pw_s, [jnp.full((16,), t, jnp.int32) + v * TPAD])
                vbase = jnp.full((16,), 0, jnp.int32) + v * VPAD

                @plsc.parallel_loop(0, B // 16, unroll=8)
                def _cols(c):
                    idxv = idx_v[pl.ds(t * B + c * 16, 16)]
                    gv = plsc.load_gather(twt_all, [vbase + idxv])
                    buf[v, pl.ds(c * 16, 16)] = gv + pwv

                return _

            lax.fori_loop(0, _VPT, v_body, None)
            _start(buf, t)

    for k in range(2):
        _wait(bufs[k])


def kernel(indices, tok_table, pos_table, W, b):
    Bv, Tv = indices.shape
    idx_tmaj = indices.T.reshape(-1).astype(jnp.int32)
    tok_pad = jnp.pad(tok_table, ((0, VPAD - VOCAB), (0, 0)))
    pos_pad = jnp.pad(pos_table[:Tv], ((0, TPAD - Tv), (0, 0)))
    W_pv = jnp.pad(W, ((0, 0), (0, VPAD - VOCAB)))
    b_col = jnp.pad(b, (0, VPAD - VOCAB)).reshape(VPAD, 1)
    twt, pwt = _tc_tables(tok_pad, pos_pad, W_pv, b_col)
    out3 = _sc_emit(idx_tmaj, twt.reshape(-1), pwt.reshape(-1))
    return jnp.transpose(out3, (2, 0, 1))


# speedup vs baseline: 9.0094x; 2.9365x over previous
import functools

import jax
import jax.numpy as jnp
from jax import lax
from jax.experimental import pallas as pl
from jax.experimental.pallas import tpu as pltpu
from jax.experimental.pallas import tpu_sc as plsc

VOCAB = 1000
N_EMBD = 64
D_PAD = 128
T = 20
B = 1024
NROWS = B * T

_NC = 2
_NS = 16
_NW = _NC * _NS
_RPW = NROWS // _NW


_HALF = _RPW // 2


@functools.partial(
    pl.kernel,
    mesh=plsc.VectorSubcoreMesh(core_axis_name="c", subcore_axis_name="s"),
    out_type=jax.ShapeDtypeStruct((NROWS, D_PAD), jnp.float32),
    scratch_types=[
        pltpu.VMEM((_RPW,), jnp.int32),
        [pltpu.VMEM((_HALF, D_PAD), jnp.float32) for _ in range(2)],
        [pltpu.SemaphoreType.DMA for _ in range(4)],
    ],
)
def _sc_gather(idx_hbm, table_hbm, out_hbm, idx_v, rows, sems):
    wid = lax.axis_index("s") * _NC + lax.axis_index("c")
    base = wid * _RPW
    pltpu.sync_copy(idx_hbm.at[pl.ds(base, _RPW)], idx_v)
    g0 = pltpu.async_copy(table_hbm.at[idx_v.at[pl.ds(0, _HALF)]],
                          rows[0], sems[0])
    g1 = pltpu.async_copy(table_hbm.at[idx_v.at[pl.ds(_HALF, _HALF)]],
                          rows[1], sems[1])
    g0.wait()
    s0 = pltpu.async_copy(rows[0], out_hbm.at[pl.ds(base, _HALF)], sems[2])
    g1.wait()
    s1 = pltpu.async_copy(rows[1], out_hbm.at[pl.ds(base + _HALF, _HALF)],
                          sems[3])
    s0.wait()
    s1.wait()


_TBLK = 4


def _tc_body(x_ref, pos_ref, w_ref, b_ref, out_ref):
    i = pl.program_id(0)
    for k in range(_TBLK):
        xp = x_ref[k] + pos_ref[i * _TBLK + k, :]
        acc = lax.dot_general(
            w_ref[...], xp,
            dimension_numbers=(((0,), (1,)), ((), ())),
            preferred_element_type=jnp.float32,
        )
        out_ref[k] = acc + b_ref[...]


def _tc_head(x3, pos_pad, W_pad, b_col):
    return pl.pallas_call(
        _tc_body,
        grid=(T // _TBLK,),
        in_specs=[
            pl.BlockSpec((_TBLK, B, D_PAD), lambda i: (i, 0, 0)),
            pl.BlockSpec((T, D_PAD), lambda i: (0, 0)),
            pl.BlockSpec((D_PAD, VOCAB), lambda i: (0, 0)),
            pl.BlockSpec((VOCAB, 1), lambda i: (0, 0)),
        ],
        out_specs=pl.BlockSpec((_TBLK, VOCAB, B), lambda i: (i, 0, 0)),
        out_shape=jax.ShapeDtypeStruct((T, VOCAB, B), jnp.float32),
    )(x3, pos_pad, W_pad, b_col)


def kernel(indices, tok_table, pos_table, W, b):
    Bv, Tv = indices.shape
    idx_tmaj = indices.T.reshape(-1).astype(jnp.int32)
    tok_pad = jnp.pad(tok_table, ((0, 0), (0, D_PAD - N_EMBD)))
    W_pad = jnp.pad(W, ((0, D_PAD - N_EMBD), (0, 0)))
    pos_pad = jnp.pad(pos_table[:Tv], ((0, 0), (0, D_PAD - N_EMBD)))
    x2 = _sc_gather(idx_tmaj, tok_pad)
    x3 = x2.reshape(Tv, Bv, D_PAD)
    b_col = b.reshape(VOCAB, 1)
    out3 = _tc_head(x3, pos_pad, W_pad, b_col)
    return jnp.transpose(out3, (2, 0, 1))
